# Initial kernel scaffold; baseline (speedup 1.0000x reference)
import jax
import jax.numpy as jnp
from jax.experimental import pallas as pl


def kernel(x_user, x_item, w_neigh_u2i_1, w_self_u2i_1, b_u2i_1, w_neigh_i2u_1, w_self_i2u_1, b_i2u_1, w_neigh_u2i_2, w_self_u2i_2, b_u2i_2, w_neigh_i2u_2, w_self_i2u_2, b_i2u_2, edge_index_u2i, edge_index_i2u):
    raise NotImplementedError("write your pallas kernel here")


# speedup vs baseline: 7.6601x; 7.6601x over previous
import functools

import jax
import jax.numpy as jnp
from jax import lax
from jax.experimental import pallas as pl
from jax.experimental.pallas import tpu as pltpu
from jax.experimental.pallas import tpu_sc as plsc

N_NODE = 5000
D = 128
E = 160000

NC = 2
NS = 16
K = 80
EPT = E // NS
NCH = EPT // K
ACC_ROWS = 5120
STRIPE = ACC_ROWS // NS


def _zero_vmem_2d(ref, nrows):
    def body(r, _):
        for k in range(D // 16):
            ref[r, pl.ds(k * 16, 16)] = jnp.zeros((16,), jnp.float32)
        return _
    lax.fori_loop(0, nrows, body, None)


def _zero_vmem_1d(ref, n):
    def body(k, _):
        ref[pl.ds(k * 16, 16)] = jnp.zeros((16,), jnp.float32)
        return _
    lax.fori_loop(0, n // 16, body, None)


def _agg_one_type(t, x_hbm, sidx_hbm, didx_hbm, out_s_hbm, out_c_hbm,
                  acc, cacc, rows0, rows1, sidx, didx, ones_v, zc, sem0, sem1):
    _zero_vmem_2d(rows0, K)
    base = t * STRIPE
    for i in range(STRIPE // K):
        pltpu.sync_copy(rows0, acc.at[pl.ds(base + i * K, K)])
    _zero_vmem_1d(zc, STRIPE)
    pltpu.sync_copy(zc, cacc.at[pl.ds(base, STRIPE)])
    pltpu.sync_copy(sidx_hbm.at[t], sidx)
    pltpu.sync_copy(didx_hbm.at[t], didx)
    def ob(k, _):
        ones_v[pl.ds(k * 16, 16)] = jnp.ones((16,), jnp.float32)
        return _
    lax.fori_loop(0, K // 16, ob, None)
    plsc.subcore_barrier()

    pltpu.async_copy(x_hbm.at[sidx.at[0]], rows0, sem0)

    def chunk_pair(jj, _):
        j0 = jj * 2
        pltpu.async_copy(x_hbm.at[sidx.at[j0 + 1]], rows1, sem1)
        pltpu.make_async_copy(x_hbm.at[sidx.at[j0]], rows0, sem0).wait()
        pltpu.sync_copy(rows0, acc.at[didx.at[j0]], add=True)
        pltpu.sync_copy(ones_v, cacc.at[didx.at[j0]], add=True)
        pltpu.async_copy(x_hbm.at[sidx.at[j0 + 2]], rows0, sem0)
        pltpu.make_async_copy(x_hbm.at[sidx.at[j0 + 1]], rows1, sem1).wait()
        pltpu.sync_copy(rows1, acc.at[didx.at[j0 + 1]], add=True)
        pltpu.sync_copy(ones_v, cacc.at[didx.at[j0 + 1]], add=True)
        return _

    lax.fori_loop(0, (NCH - 1) // 2, chunk_pair, None)
    pltpu.make_async_copy(x_hbm.at[sidx.at[NCH - 1]], rows0, sem0).wait()
    pltpu.sync_copy(rows0, acc.at[didx.at[NCH - 1]], add=True)
    pltpu.sync_copy(ones_v, cacc.at[didx.at[NCH - 1]], add=True)

    plsc.subcore_barrier()
    last = N_NODE - (NS - 1) * STRIPE
    pltpu.sync_copy(cacc.at[pl.ds(base, STRIPE)], zc)
    @pl.when(t < NS - 1)
    def _():
        pltpu.sync_copy(acc.at[pl.ds(base, STRIPE)],
                        out_s_hbm.at[pl.ds(base, STRIPE)])
        pltpu.sync_copy(zc, out_c_hbm.at[pl.ds(base, STRIPE)])
    @pl.when(t == NS - 1)
    def _():
        pltpu.sync_copy(acc.at[pl.ds(base, last)],
                        out_s_hbm.at[pl.ds(base, last)])
        pltpu.sync_copy(zc.at[pl.ds(0, last)], out_c_hbm.at[pl.ds(base, last)])


_SC_MESH = plsc.VectorSubcoreMesh(
    core_axis_name="c", subcore_axis_name="s", num_cores=NC, num_subcores=NS)

_F32 = jnp.float32


@functools.partial(
    pl.kernel,
    out_type=[
        jax.ShapeDtypeStruct((N_NODE, D), _F32),
        jax.ShapeDtypeStruct((N_NODE, D), _F32),
        jax.ShapeDtypeStruct((N_NODE,), _F32),
        jax.ShapeDtypeStruct((N_NODE,), _F32),
    ],
    mesh=_SC_MESH,
    scratch_types=[
        pltpu.VMEM_SHARED((ACC_ROWS, D), _F32),
        pltpu.VMEM_SHARED((ACC_ROWS,), _F32),
        pltpu.VMEM((K, D), _F32),
        pltpu.VMEM((K, D), _F32),
        pltpu.VMEM((NCH, K), jnp.int32),
        pltpu.VMEM((NCH, K), jnp.int32),
        pltpu.VMEM((K,), _F32),
        pltpu.VMEM((STRIPE,), _F32),
        pltpu.SemaphoreType.DMA,
        pltpu.SemaphoreType.DMA,
    ],
)
def _sc_aggregate(x_user_hbm, x_item_hbm,
                  sidx_u2i, didx_u2i, sidx_i2u, didx_i2u,
                  s_item_hbm, s_user_hbm, c_item_hbm, c_user_hbm,
                  acc, cacc, rows0, rows1, sidx, didx, ones_v, zc,
                  sem0, sem1):
    c = lax.axis_index("c")
    t = lax.axis_index("s")

    @pl.when(c == 0)
    def _():
        _agg_one_type(t, x_user_hbm, sidx_u2i, didx_u2i,
                      s_item_hbm, c_item_hbm,
                      acc, cacc, rows0, rows1, sidx, didx, ones_v, zc,
                      sem0, sem1)

    @pl.when(c == 1)
    def _():
        _agg_one_type(t, x_item_hbm, sidx_i2u, didx_i2u,
                      s_user_hbm, c_user_hbm,
                      acc, cacc, rows0, rows1, sidx, didx, ones_v, zc,
                      sem0, sem1)


BLK = 1000


def _linear_body(relu, s_ref, c_ref, x_ref, wn_ref, ws_ref, b_ref, o_ref):
    cnt = jnp.maximum(c_ref[...], 1.0)
    agg = s_ref[...] / cnt
    y = jnp.dot(agg, wn_ref[...], preferred_element_type=jnp.float32)
    y = y + jnp.dot(x_ref[...], ws_ref[...], preferred_element_type=jnp.float32)
    y = y + b_ref[...]
    if relu:
        y = jnp.maximum(y, 0.0)
    o_ref[...] = y


def _tc_linear(s, cnt, x, w_neigh, w_self, b, relu):
    grid = (N_NODE // BLK,)
    return pl.pallas_call(
        functools.partial(_linear_body, relu),
        grid=grid,
        in_specs=[
            pl.BlockSpec((BLK, D), lambda i: (i, 0)),
            pl.BlockSpec((BLK, 1), lambda i: (i, 0)),
            pl.BlockSpec((BLK, D), lambda i: (i, 0)),
            pl.BlockSpec((D, D), lambda i: (0, 0)),
            pl.BlockSpec((D, D), lambda i: (0, 0)),
            pl.BlockSpec((1, D), lambda i: (0, 0)),
        ],
        out_specs=pl.BlockSpec((BLK, D), lambda i: (i, 0)),
        out_shape=jax.ShapeDtypeStruct((N_NODE, D), jnp.float32),
    )(s, cnt.reshape(N_NODE, 1), x, w_neigh, w_self, b.reshape(1, D))


def kernel(x_user, x_item, w_neigh_u2i_1, w_self_u2i_1, b_u2i_1,
           w_neigh_i2u_1, w_self_i2u_1, b_i2u_1,
           w_neigh_u2i_2, w_self_u2i_2, b_u2i_2,
           w_neigh_i2u_2, w_self_i2u_2, b_i2u_2,
           edge_index_u2i, edge_index_i2u):
    su = edge_index_u2i.astype(jnp.int32).reshape(2, NS, NCH, K)
    si = edge_index_i2u.astype(jnp.int32).reshape(2, NS, NCH, K)

    s_item, s_user, c_item, c_user = _sc_aggregate(
        x_user, x_item, su[0], su[1], si[0], si[1])
    h_item = _tc_linear(s_item, c_item, x_item,
                        w_neigh_u2i_1, w_self_u2i_1, b_u2i_1, relu=True)
    h_user = _tc_linear(s_user, c_user, x_user,
                        w_neigh_i2u_1, w_self_i2u_1, b_i2u_1, relu=True)

    s_item2, s_user2, c_item2, c_user2 = _sc_aggregate(
        h_user, h_item, su[0], su[1], si[0], si[1])
    o_item = _tc_linear(s_item2, c_item2, h_item,
                        w_neigh_u2i_2, w_self_u2i_2, b_u2i_2, relu=False)
    o_user = _tc_linear(s_user2, c_user2, h_user,
                        w_neigh_i2u_2, w_self_i2u_2, b_i2u_2, relu=False)
    return jnp.concatenate([o_user, o_item], axis=0)
